# baseline (speedup 1.0000x reference)
import jax
import jax.numpy as jnp
from jax import lax
from jax.experimental import pallas as pl
from jax.experimental.pallas import tpu as pltpu
from jax.experimental.pallas import tpu_sc as plsc

N = 40000
M = 32
C_IN = 4
H = 496
W = 432
B = 4
HW = H * W
CELLS = B * HW
C_OUT = 5
OUT_LEN = CELLS * C_OUT
FW = 16


N_PAD = 40960
_TC_BLK = 5120
_TC_GRID = N_PAD // _TC_BLK


def _feat_body(vox_ref, npf_ref, coordsf_ref,
               f0_ref, f1_ref, f2_ref, f3_ref, f4_ref, idx_ref):
    x = vox_ref[...]
    rmod = lax.broadcasted_iota(jnp.int32, (FW, 128), 1) % C_IN
    srow = lax.broadcasted_iota(jnp.int32, (FW, 128), 0)
    selt = ((rmod + 1 == srow) & (rmod < 3)).astype(jnp.float32)
    s2 = lax.dot_general(selt, x, (((1,), (1,)), ((), ())),
                         preferred_element_type=jnp.float32)
    npr = npf_ref[...]
    inv = 1.0 / npr
    m1 = s2[1:2, :] * inv
    m2 = s2[2:3, :] * inv
    m3 = s2[3:4, :] * inv
    d = jnp.sqrt(m1 * m1 + m2 * m2 + m3 * m3)
    ct = coordsf_ref[...]
    idxr = ct[0:1, :] * float(HW) + ct[2:3, :] * float(W) + ct[3:4, :]
    f0_ref[...] = npr[0, :]
    f1_ref[...] = m1[0, :]
    f2_ref[...] = m2[0, :]
    f3_ref[...] = m3[0, :]
    f4_ref[...] = d[0, :]
    idx_ref[...] = idxr[0, :].astype(jnp.int32)


def _feat_stage(vox2d, npf1, coordsf):
    return pl.pallas_call(
        _feat_body,
        grid=(_TC_GRID,),
        in_specs=[
            pl.BlockSpec((_TC_BLK, 128), lambda i: (i, 0)),
            pl.BlockSpec((1, _TC_BLK), lambda i: (0, i)),
            pl.BlockSpec((4, _TC_BLK), lambda i: (0, i)),
        ],
        out_specs=[pl.BlockSpec((_TC_BLK,), lambda i: (i,))] * 6,
        out_shape=[jax.ShapeDtypeStruct((N_PAD,), jnp.float32)] * 5
        + [jax.ShapeDtypeStruct((N_PAD,), jnp.int32)],
    )(vox2d, npf1, coordsf)


IDX_CH = 2000
N_IDX_CH = N // IDX_CH
UNROLL = 5
GPC = IDX_CH // (16 * UNROLL)
ROWS_CH = 256
Q_ROWS = 16
Q_CELLS = Q_ROWS * W
AUX_CELLS = 64 * W


def _scatter_body(f0_hbm, f1_hbm, f2_hbm, f3_hbm, f4_hbm, idx_hbm, out_hbm,
                  ib0, ib1, aux, ids, pos, o0, o1, o2, o3, o4,
                  r0b, r1b, r2b, r3b, r4b, sem):
    info = plsc.get_sparse_core_info()
    nc = info.num_cores
    nw = nc * info.num_subcores
    wpp = nw // B
    fc = [f0_hbm, f1_hbm, f2_hbm, f3_hbm, f4_hbm]
    outc = [o0, o1, o2, o3, o4]
    rowb = [r0b, r1b, r2b, r3b, r4b]
    ibuf = [ib0, ib1]
    wid = lax.axis_index("s") * nc + lax.axis_index("c")
    plane = wid // wpp
    j = wid % wpp
    iota = lax.iota(jnp.int32, 16)
    zf = jnp.zeros((16,), jnp.float32)
    zi = jnp.zeros((16,), jnp.int32)

    row0 = jnp.where(j < 6, 64 * j, 384 + 56 * (j - 6))
    nrows = jnp.where(j < 6, 64, 56)
    lo = (plane * H + row0) * W
    hi = lo + nrows * W

    def zero_body(i, _):
        for k in range(4):
            aux[pl.ds(i * 64 + k * 16, 16)] = zi
        return 0
    lax.fori_loop(0, AUX_CELLS // 64, zero_body, 0)

    descs = []
    descs.append(pltpu.async_copy(idx_hbm.at[pl.ds(0, IDX_CH)], ibuf[0], sem))
    for ch in range(N_IDX_CH):
        if ch + 1 < N_IDX_CH:
            descs.append(pltpu.async_copy(
                idx_hbm.at[pl.ds((ch + 1) * IDX_CH, IDX_CH)], ibuf[(ch + 1) % 2], sem))
        descs[ch].wait()
        buf = ibuf[ch % 2]

        def p1_body(g, _, ch=ch, buf=buf):
            base = g * (16 * UNROLL)
            nvb = ch * IDX_CH + base + 1
            ivs = [buf[pl.ds(base + k * 16, 16)] for k in range(UNROLL)]
            inrs = [(iv >= lo) & (iv < hi) for iv in ivs]
            lasts = [plsc.scan_count(iv, mask=inr)[1]
                     for iv, inr in zip(ivs, inrs)]
            for k in range(UNROLL):
                m = inrs[k] & lasts[k]
                loc = jnp.where(m, ivs[k] - lo, 0)
                nv = iota + (nvb + k * 16)
                plsc.store_scatter(aux, [loc], nv, mask=m)
            return 0
        lax.fori_loop(0, GPC, p1_body, 0)

    def chunk_pipeline(rows, abase, y0):
        cells = rows * W

        def zout_body(r, _):
            for cc in range(W // 16):
                for c in range(C_OUT):
                    outc[c][r, pl.ds(cc * 16, 16)] = zf
            return 0
        lax.fori_loop(0, rows, zout_body, 0)

        def p2_body(w, off):
            av0 = aux[pl.ds(abase + w * 32, 16)]
            av1 = aux[pl.ds(abase + w * 32 + 16, 16)]
            m0 = av0 > 0
            m1 = av1 > 0
            plsc.store_compressed(ids.at[pl.ds(off, 16)], av0 - 1, mask=m0)
            plsc.store_compressed(pos.at[pl.ds(off, 16)], w * 32 + iota, mask=m0)
            off1 = off + jnp.sum(jnp.where(m0, 1, 0))
            plsc.store_compressed(ids.at[pl.ds(off1, 16)], av1 - 1, mask=m1)
            plsc.store_compressed(pos.at[pl.ds(off1, 16)], w * 32 + 16 + iota, mask=m1)
            return off1 + jnp.sum(jnp.where(m1, 1, 0))
        cnt = lax.fori_loop(0, cells // 32, p2_body, 0)

        nch = (cnt + ROWS_CH - 1) // ROWS_CH

        m0 = iota == 0
        id0 = plsc.cummax(jnp.where(m0, ids[pl.ds(0, 16)], -1))
        pos0 = plsc.cummax(jnp.where(m0, pos[pl.ds(0, 16)], -1))

        def pad_body(w, _):
            flat = w * 16 + iota
            m = flat >= cnt
            plsc.store_scatter(ids, [flat], id0, mask=m)
            plsc.store_scatter(pos, [flat], pos0, mask=m)
            return 0
        lax.fori_loop(cnt // 16, nch * (ROWS_CH // 16), pad_body, 0)

        def g_cond(ci):
            return ci < nch

        def g_body(ci):
            ds_ = [
                pltpu.async_copy(
                    fc[c].at[ids.at[pl.ds(ci * ROWS_CH, ROWS_CH)]], rowb[c], sem
                )
                for c in range(C_OUT)
            ]
            for d in ds_:
                d.wait()

            def d_body(w, _):
                pv = pos[pl.ds(ci * ROWS_CH + w * 16, 16)]
                pr = pv // W
                pc2 = pv - pr * W
                for c in range(C_OUT):
                    rv = rowb[c][pl.ds(w * 16, 16)]
                    plsc.store_scatter(outc[c], [pr, pc2], rv)
                return 0
            lax.fori_loop(0, ROWS_CH // 16, d_body, 0)
            return ci + 1
        lax.while_loop(g_cond, g_body, 0)

        wdescs = []
        for c in range(C_OUT):
            for u in range(rows // 8):
                wdescs.append(pltpu.async_copy(
                    outc[c].at[pl.ds(u * 8, 8), :],
                    out_hbm.at[pl.ds((plane * C_OUT + c) * H + y0 + u * 8, 8), :],
                    sem))
        for d in wdescs:
            d.wait()

    nq16 = jnp.where(j < 6, 4, 3)

    def q_body(q, _):
        chunk_pipeline(Q_ROWS, q * Q_CELLS, row0 + q * Q_ROWS)
        return 0
    lax.fori_loop(0, nq16, q_body, 0)

    @pl.when(j >= 6)
    def _():
        chunk_pipeline(8, 3 * Q_CELLS, row0 + 48)


def _scatter_stage(f0, f1, f2, f3, f4, idx):
    mesh = plsc.VectorSubcoreMesh(core_axis_name="c", subcore_axis_name="s")
    f = pl.kernel(
        _scatter_body,
        out_type=jax.ShapeDtypeStruct((B * C_OUT * H, W), jnp.float32),
        mesh=mesh,
        compiler_params=pltpu.CompilerParams(needs_layout_passes=False),
        scratch_types=[
            pltpu.VMEM((IDX_CH,), jnp.int32),
            pltpu.VMEM((IDX_CH,), jnp.int32),
            pltpu.VMEM((AUX_CELLS,), jnp.int32),
            pltpu.VMEM((Q_CELLS,), jnp.int32),
            pltpu.VMEM((Q_CELLS,), jnp.int32),
        ] + [pltpu.VMEM((Q_ROWS, W), jnp.float32) for _ in range(C_OUT)]
        + [pltpu.VMEM((ROWS_CH,), jnp.float32) for _ in range(C_OUT)]
        + [pltpu.SemaphoreType.DMA],
    )
    return f(f0, f1, f2, f3, f4, idx)


def kernel(voxels, voxel_num_points, voxel_coords):
    vox2d = voxels.reshape(N, M * C_IN)
    npf1 = voxel_num_points.astype(jnp.float32).reshape(1, N)
    coordsf = voxel_coords.astype(jnp.float32).T
    f0, f1, f2, f3, f4, idx = _feat_stage(vox2d, npf1, coordsf)
    out2d = _scatter_stage(f0, f1, f2, f3, f4, idx)
    return out2d.reshape(B, C_OUT, H, W)


# speedup vs baseline: 1.0073x; 1.0073x over previous
import jax
import jax.numpy as jnp
from jax import lax
from jax.experimental import pallas as pl
from jax.experimental.pallas import tpu as pltpu
from jax.experimental.pallas import tpu_sc as plsc

N = 40000
M = 32
C_IN = 4
H = 496
W = 432
B = 4
HW = H * W
CELLS = B * HW
C_OUT = 5
OUT_LEN = CELLS * C_OUT
FW = 16


N_PAD = 40960
_TC_BLK = 5120
_TC_GRID = N_PAD // _TC_BLK


def _feat_body(vox_ref, npf_ref, coordsf_ref,
               f0_ref, f1_ref, f2_ref, f3_ref, f4_ref, idx_ref):
    x = vox_ref[...]
    rmod = lax.broadcasted_iota(jnp.int32, (FW, 128), 1) % C_IN
    srow = lax.broadcasted_iota(jnp.int32, (FW, 128), 0)
    selt = ((rmod + 1 == srow) & (rmod < 3)).astype(jnp.float32)
    s2 = lax.dot_general(selt, x, (((1,), (1,)), ((), ())),
                         preferred_element_type=jnp.float32)
    npr = npf_ref[...]
    inv = 1.0 / npr
    m1 = s2[1:2, :] * inv
    m2 = s2[2:3, :] * inv
    m3 = s2[3:4, :] * inv
    d = jnp.sqrt(m1 * m1 + m2 * m2 + m3 * m3)
    ct = coordsf_ref[...]
    idxr = ct[0:1, :] * float(HW) + ct[2:3, :] * float(W) + ct[3:4, :]
    f0_ref[...] = npr[0, :]
    f1_ref[...] = m1[0, :]
    f2_ref[...] = m2[0, :]
    f3_ref[...] = m3[0, :]
    f4_ref[...] = d[0, :]
    idx_ref[...] = idxr[0, :].astype(jnp.int32)


def _feat_stage(vox2d, npf1, coordsf):
    return pl.pallas_call(
        _feat_body,
        grid=(_TC_GRID,),
        in_specs=[
            pl.BlockSpec((_TC_BLK, 128), lambda i: (i, 0)),
            pl.BlockSpec((1, _TC_BLK), lambda i: (0, i)),
            pl.BlockSpec((4, _TC_BLK), lambda i: (0, i)),
        ],
        out_specs=[pl.BlockSpec((_TC_BLK,), lambda i: (i,))] * 6,
        out_shape=[jax.ShapeDtypeStruct((N_PAD,), jnp.float32)] * 5
        + [jax.ShapeDtypeStruct((N_PAD,), jnp.int32)],
    )(vox2d, npf1, coordsf)


IDX_CH = 1600
N_IDX_CH = N // IDX_CH
UNROLL = 5
GPC = IDX_CH // (16 * UNROLL)
ROWS_CH = 256
Q_ROWS = 16
Q_CELLS = Q_ROWS * W
AUX_CELLS = 64 * W


def _scatter_body(f0_hbm, f1_hbm, f2_hbm, f3_hbm, f4_hbm, idx_hbm, out_hbm,
                  ib0, ib1, aux, ids, pos, o0, o1, o2, o3, o4,
                  r0b, r1b, r2b, r3b, r4b, sem):
    info = plsc.get_sparse_core_info()
    nc = info.num_cores
    nw = nc * info.num_subcores
    wpp = nw // B
    fc = [f0_hbm, f1_hbm, f2_hbm, f3_hbm, f4_hbm]
    outc = [o0, o1, o2, o3, o4]
    rowb = [r0b, r1b, r2b, r3b, r4b]
    ibuf = [ib0, ib1]
    wid = lax.axis_index("s") * nc + lax.axis_index("c")
    plane = wid // wpp
    j = wid % wpp
    iota = lax.iota(jnp.int32, 16)
    zf = jnp.zeros((16,), jnp.float32)
    zi = jnp.zeros((16,), jnp.int32)

    row0 = jnp.where(j < 6, 64 * j, 384 + 56 * (j - 6))
    nrows = jnp.where(j < 6, 64, 56)
    lo = (plane * H + row0) * W
    hi = lo + nrows * W

    def zero_body(i, _):
        for k in range(4):
            aux[pl.ds(i * 64 + k * 16, 16)] = zi
        return 0
    lax.fori_loop(0, AUX_CELLS // 64, zero_body, 0)

    descs = []
    descs.append(pltpu.async_copy(idx_hbm.at[pl.ds(0, IDX_CH)], ibuf[0], sem))
    for ch in range(N_IDX_CH):
        if ch + 1 < N_IDX_CH:
            descs.append(pltpu.async_copy(
                idx_hbm.at[pl.ds((ch + 1) * IDX_CH, IDX_CH)], ibuf[(ch + 1) % 2], sem))
        descs[ch].wait()
        buf = ibuf[ch % 2]

        def p1_body(g, _, ch=ch, buf=buf):
            base = g * (16 * UNROLL)
            nvb = ch * IDX_CH + base + 1
            ivs = [buf[pl.ds(base + k * 16, 16)] for k in range(UNROLL)]
            inrs = [(iv >= lo) & (iv < hi) for iv in ivs]
            lasts = [plsc.scan_count(iv, mask=inr)[1]
                     for iv, inr in zip(ivs, inrs)]
            for k in range(UNROLL):
                m = inrs[k] & lasts[k]
                loc = jnp.where(m, ivs[k] - lo, 0)
                nv = iota + (nvb + k * 16)
                plsc.store_scatter(aux, [loc], nv, mask=m)
            return 0
        lax.fori_loop(0, GPC, p1_body, 0)

    def chunk_pipeline(rows, abase, y0):
        cells = rows * W

        def zout_body(r, _):
            for cc in range(W // 16):
                for c in range(C_OUT):
                    outc[c][r, pl.ds(cc * 16, 16)] = zf
            return 0
        lax.fori_loop(0, rows, zout_body, 0)

        def p2_body(w, off):
            av = aux[pl.ds(abase + w * 16, 16)]
            m = av > 0
            plsc.store_compressed(ids.at[pl.ds(off, 16)], av - 1, mask=m)
            plsc.store_compressed(pos.at[pl.ds(off, 16)], w * 16 + iota, mask=m)
            return off + jnp.sum(jnp.where(m, 1, 0))
        cnt = lax.fori_loop(0, cells // 16, p2_body, 0)

        nch = (cnt + ROWS_CH - 1) // ROWS_CH

        m0 = iota == 0
        id0 = plsc.cummax(jnp.where(m0, ids[pl.ds(0, 16)], -1))
        pos0 = plsc.cummax(jnp.where(m0, pos[pl.ds(0, 16)], -1))

        def pad_body(w, _):
            flat = w * 16 + iota
            m = flat >= cnt
            plsc.store_scatter(ids, [flat], id0, mask=m)
            plsc.store_scatter(pos, [flat], pos0, mask=m)
            return 0
        lax.fori_loop(cnt // 16, nch * (ROWS_CH // 16), pad_body, 0)

        def g_cond(ci):
            return ci < nch

        def g_body(ci):
            ds_ = [
                pltpu.async_copy(
                    fc[c].at[ids.at[pl.ds(ci * ROWS_CH, ROWS_CH)]], rowb[c], sem
                )
                for c in range(C_OUT)
            ]
            for d in ds_:
                d.wait()

            def d_body(w, _):
                pv = pos[pl.ds(ci * ROWS_CH + w * 16, 16)]
                pr = pv // W
                pc2 = pv - pr * W
                for c in range(C_OUT):
                    rv = rowb[c][pl.ds(w * 16, 16)]
                    plsc.store_scatter(outc[c], [pr, pc2], rv)
                return 0
            lax.fori_loop(0, ROWS_CH // 16, d_body, 0)
            return ci + 1
        lax.while_loop(g_cond, g_body, 0)

        wdescs = []
        for c in range(C_OUT):
            for u in range(rows // 8):
                wdescs.append(pltpu.async_copy(
                    outc[c].at[pl.ds(u * 8, 8), :],
                    out_hbm.at[pl.ds((plane * C_OUT + c) * H + y0 + u * 8, 8), :],
                    sem))
        for d in wdescs:
            d.wait()

    nq16 = jnp.where(j < 6, 4, 3)

    def q_body(q, _):
        chunk_pipeline(Q_ROWS, q * Q_CELLS, row0 + q * Q_ROWS)
        return 0
    lax.fori_loop(0, nq16, q_body, 0)

    @pl.when(j >= 6)
    def _():
        chunk_pipeline(8, 3 * Q_CELLS, row0 + 48)


def _scatter_stage(f0, f1, f2, f3, f4, idx):
    mesh = plsc.VectorSubcoreMesh(core_axis_name="c", subcore_axis_name="s")
    f = pl.kernel(
        _scatter_body,
        out_type=jax.ShapeDtypeStruct((B * C_OUT * H, W), jnp.float32),
        mesh=mesh,
        compiler_params=pltpu.CompilerParams(needs_layout_passes=False),
        scratch_types=[
            pltpu.VMEM((IDX_CH,), jnp.int32),
            pltpu.VMEM((IDX_CH,), jnp.int32),
            pltpu.VMEM((AUX_CELLS,), jnp.int32),
            pltpu.VMEM((Q_CELLS,), jnp.int32),
            pltpu.VMEM((Q_CELLS,), jnp.int32),
        ] + [pltpu.VMEM((Q_ROWS, W), jnp.float32) for _ in range(C_OUT)]
        + [pltpu.VMEM((ROWS_CH,), jnp.float32) for _ in range(C_OUT)]
        + [pltpu.SemaphoreType.DMA],
    )
    return f(f0, f1, f2, f3, f4, idx)


def kernel(voxels, voxel_num_points, voxel_coords):
    vox2d = voxels.reshape(N, M * C_IN)
    npf1 = voxel_num_points.astype(jnp.float32).reshape(1, N)
    coordsf = voxel_coords.astype(jnp.float32).T
    f0, f1, f2, f3, f4, idx = _feat_stage(vox2d, npf1, coordsf)
    out2d = _scatter_stage(f0, f1, f2, f3, f4, idx)
    return out2d.reshape(B, C_OUT, H, W)
